# baseline (speedup 1.0000x reference)
import functools

import jax
import jax.numpy as jnp
from jax import lax
from jax.experimental import pallas as pl
from jax.experimental.pallas import tpu as pltpu
from jax.experimental.pallas import tpu_sc as plsc

N_NODES = 50000
N_EDGES = 3200000
N_GRAPHS = 512

SUB = 128
NSUB = 8
CHUNK = SUB * NSUB
NW = 32
NBUF = 4
G_ITERS_A = 76
G_ITERS_B = 120
E_PAD = 16 * (G_ITERS_A + G_ITERS_B) * CHUNK
N_PAD = 50048
ROWS_TBL = N_PAD
ROWS_PER_TILE = 3128

ROW_BLK = 2000
TC_GRID = N_NODES // ROW_BLK


def _sc_body(do_gather, nq, src1, dst1, *refs):
    hs_l = refs[:nq]
    out_l = refs[nq:2 * nq]
    idx_s, idx_d, rows, shared, sem_i, sem_g, sem_s = refs[2 * nq:]
    c = lax.axis_index("c")
    s = lax.axis_index("s")
    r0 = s * ROWS_PER_TILE
    gcur = G_ITERS_A + c * (G_ITERS_B - G_ITERS_A)
    base = (c * 16 * G_ITERS_A + s * gcur) * CHUNK

    for q in range(nq):
        hs = hs_l[q]
        pltpu.sync_copy(hs.at[pl.ds(r0, ROWS_PER_TILE)],
                        shared.at[pl.ds(r0, ROWS_PER_TILE)])
        if not do_gather:
            pltpu.sync_copy(hs.at[pl.ds(0, CHUNK)], rows.at[0])
        plsc.subcore_barrier()

        def fire_idx(g):
            b = lax.rem(g, NBUF)
            e0 = base + g * CHUNK
            pltpu.async_copy(dst1.at[pl.ds(e0, CHUNK)], idx_d.at[b],
                             sem_i.at[b])
            if do_gather:
                pltpu.async_copy(src1.at[pl.ds(e0, CHUNK)], idx_s.at[b],
                                 sem_i.at[b])

        def wait_idx(g):
            b = lax.rem(g, NBUF)
            pltpu.make_async_copy(dst1.at[pl.ds(0, CHUNK)], idx_d.at[b],
                                  sem_i.at[b]).wait()
            if do_gather:
                pltpu.make_async_copy(src1.at[pl.ds(0, CHUNK)], idx_s.at[b],
                                      sem_i.at[b]).wait()

        def fire_gather(g):
            if do_gather:
                b = lax.rem(g, NBUF)
                pltpu.async_copy(hs.at[idx_s.at[b]], rows.at[b], sem_g.at[b])

        def wait_gather(g):
            if do_gather:
                b = lax.rem(g, NBUF)
                pltpu.make_async_copy(hs.at[idx_s.at[b]], rows.at[b],
                                      sem_g.at[b]).wait()

        def fire_scatter(g):
            b = lax.rem(g, NBUF)
            rb = b if do_gather else 0
            pltpu.async_copy(rows.at[rb], shared.at[idx_d.at[b]],
                             sem_s.at[b], add=True)

        def wait_scatter(g):
            b = lax.rem(g, NBUF)
            rb = b if do_gather else 0
            pltpu.make_async_copy(rows.at[rb], shared.at[idx_d.at[b]],
                                  sem_s.at[b]).wait()

        fire_idx(0)
        fire_idx(1)

        def body(g, carry):
            wait_idx(g)
            fire_gather(g)

            @pl.when(g >= 1)
            def _():
                wait_gather(g - 1)
                fire_scatter(g - 1)

            @pl.when(g >= 2)
            def _():
                wait_scatter(g - 2)

            @pl.when(g + 2 < gcur)
            def _():
                fire_idx(g + 2)

            return carry

        lax.fori_loop(0, gcur, body, 0)
        wait_gather(gcur - 1)
        fire_scatter(gcur - 1)
        wait_scatter(gcur - 2)
        wait_scatter(gcur - 1)
        plsc.subcore_barrier()
        pltpu.sync_copy(shared.at[pl.ds(r0, ROWS_PER_TILE)],
                        out_l[q].at[c].at[pl.ds(r0, ROWS_PER_TILE)])


def _sc_prop(hs_list, src2, dst2, fsc, do_gather=True):
    nq = len(hs_list)
    mesh = plsc.VectorSubcoreMesh(core_axis_name="c", subcore_axis_name="s")
    return pl.kernel(
        functools.partial(_sc_body, do_gather, nq),
        out_type=[jax.ShapeDtypeStruct((2, ROWS_TBL, fsc), jnp.float32)] * nq,
        mesh=mesh,
        scratch_types=[
            pltpu.VMEM((NBUF, CHUNK), jnp.int32),
            pltpu.VMEM((NBUF, CHUNK), jnp.int32),
            pltpu.VMEM((NBUF, CHUNK, fsc), jnp.float32),
            pltpu.VMEM_SHARED((ROWS_TBL, fsc), jnp.float32),
            pltpu.SemaphoreType.DMA((NBUF,)),
            pltpu.SemaphoreType.DMA((NBUF,)),
            pltpu.SemaphoreType.DMA((NBUF,)),
        ],
        compiler_params=pltpu.CompilerParams(use_tc_tiling_on_sc=False),
    )(src2, dst2, *hs_list)


def _prep_body(pdeg_ref, xp_ref, dinv_ref, hs0_ref):
    deg = pdeg_ref[0, :, 0:1] + pdeg_ref[1, :, 0:1] - 1.0
    dinv = lax.rsqrt(deg)
    dinv_ref[...] = dinv
    hs0_ref[...] = xp_ref[...] * dinv


def _tc_prep(pdeg, xp):
    return pl.pallas_call(
        _prep_body,
        grid=(TC_GRID,),
        in_specs=[
            pl.BlockSpec((2, ROW_BLK, 8), lambda i: (0, i, 0)),
            pl.BlockSpec((ROW_BLK, 16), lambda i: (i, 0)),
        ],
        out_specs=[
            pl.BlockSpec((ROW_BLK, 1), lambda i: (i, 0)),
            pl.BlockSpec((ROW_BLK, 16), lambda i: (i, 0)),
        ],
        out_shape=[
            jax.ShapeDtypeStruct((N_NODES, 1), jnp.float32),
            jax.ShapeDtypeStruct((N_PAD, 16), jnp.float32),
        ],
    )(pdeg, xp)


def _layer_body(nout, p_ref, hs_ref, dinv_ref, w_ref, b_ref, *o_refs):
    dinv = dinv_ref[...]
    g = dinv * (p_ref[0] + p_ref[1] - hs_ref[...])
    h = jax.nn.relu(
        jnp.dot(g, w_ref[...], preferred_element_type=jnp.float32)
        + b_ref[...])
    hs_next = h * dinv
    if nout == 1:
        o_refs[0][...] = hs_next
    else:
        for q in range(nout):
            o_refs[q][...] = hs_next[:, 16 * q:16 * (q + 1)]


def _tc_layer(p, hs, dinv, W, b, split):
    fsc = hs.shape[1]
    fout = W.shape[1]
    if split:
        nout = fout // 16
        out_specs = [pl.BlockSpec((ROW_BLK, 16), lambda i: (i, 0))] * nout
        out_shape = [jax.ShapeDtypeStruct((N_PAD, 16), jnp.float32)] * nout
    else:
        nout = 1
        out_specs = [pl.BlockSpec((ROW_BLK, fout), lambda i: (i, 0))]
        out_shape = [jax.ShapeDtypeStruct((N_PAD, fout), jnp.float32)]
    return pl.pallas_call(
        functools.partial(_layer_body, nout),
        grid=(TC_GRID,),
        in_specs=[
            pl.BlockSpec((2, ROW_BLK, fsc), lambda i: (0, i, 0)),
            pl.BlockSpec((ROW_BLK, fsc), lambda i: (i, 0)),
            pl.BlockSpec((ROW_BLK, 1), lambda i: (i, 0)),
            pl.BlockSpec((fsc, fout), lambda i: (0, 0)),
            pl.BlockSpec((1, fout), lambda i: (0, 0)),
        ],
        out_specs=out_specs,
        out_shape=out_shape,
    )(p, hs, dinv, W, b.reshape(1, fout))


def _layer3_body(p0_ref, p1_ref, p2_ref, p3_ref, hs0_ref, hs1_ref, hs2_ref,
                 hs3_ref, dinv_ref, w_ref, b_ref,
                 batch_ref, wl_ref, bl_ref, sums_ref, cnt_ref, o_ref):
    i = pl.program_id(0)
    dinv = dinv_ref[...]
    w = w_ref[...]
    p_refs = (p0_ref, p1_ref, p2_ref, p3_ref)
    hs_refs = (hs0_ref, hs1_ref, hs2_ref, hs3_ref)
    acc = b_ref[...]
    for q in range(4):
        gq = dinv * (p_refs[q][0] + p_refs[q][1] - hs_refs[q][...])
        acc = acc + jnp.dot(gq, w[16 * q:16 * (q + 1)],
                            preferred_element_type=jnp.float32)
    h3 = jax.nn.relu(acc)
    b_ids = batch_ref[0, 0, :]
    onehot = jnp.where(
        b_ids[:, None] == lax.broadcasted_iota(jnp.int32, (ROW_BLK, N_GRAPHS), 1),
        1.0, 0.0)
    part = lax.dot_general(onehot, h3, (((0,), (0,)), ((), ())),
                           preferred_element_type=jnp.float32)
    pcnt = jnp.sum(onehot, axis=0)[:, None]

    @pl.when(i == 0)
    def _():
        sums_ref[...] = jnp.zeros_like(sums_ref)
        cnt_ref[...] = jnp.zeros_like(cnt_ref)

    sums_ref[...] += part
    cnt_ref[...] += pcnt

    @pl.when(i == TC_GRID - 1)
    def _():
        pooled = sums_ref[...] / jnp.maximum(cnt_ref[...], 1.0)
        logits = (jnp.dot(pooled, wl_ref[...],
                          preferred_element_type=jnp.float32) + bl_ref[...])
        m = jnp.max(logits, axis=1, keepdims=True)
        z = logits - m
        o_ref[...] = z - jnp.log(jnp.sum(jnp.exp(z), axis=1, keepdims=True))


def _tc_layer3_pool(ps, hss, dinv, W3, b3, batch3, Wl, bl):
    return pl.pallas_call(
        _layer3_body,
        grid=(TC_GRID,),
        in_specs=[
            pl.BlockSpec((2, ROW_BLK, 16), lambda i: (0, i, 0)),
            pl.BlockSpec((2, ROW_BLK, 16), lambda i: (0, i, 0)),
            pl.BlockSpec((2, ROW_BLK, 16), lambda i: (0, i, 0)),
            pl.BlockSpec((2, ROW_BLK, 16), lambda i: (0, i, 0)),
            pl.BlockSpec((ROW_BLK, 16), lambda i: (i, 0)),
            pl.BlockSpec((ROW_BLK, 16), lambda i: (i, 0)),
            pl.BlockSpec((ROW_BLK, 16), lambda i: (i, 0)),
            pl.BlockSpec((ROW_BLK, 16), lambda i: (i, 0)),
            pl.BlockSpec((ROW_BLK, 1), lambda i: (i, 0)),
            pl.BlockSpec((64, 128), lambda i: (0, 0)),
            pl.BlockSpec((1, 128), lambda i: (0, 0)),
            pl.BlockSpec((1, 1, ROW_BLK), lambda i: (i, 0, 0)),
            pl.BlockSpec((128, 10), lambda i: (0, 0)),
            pl.BlockSpec((1, 10), lambda i: (0, 0)),
        ],
        out_specs=[
            pl.BlockSpec((N_GRAPHS, 128), lambda i: (0, 0)),
            pl.BlockSpec((N_GRAPHS, 1), lambda i: (0, 0)),
            pl.BlockSpec((N_GRAPHS, 10), lambda i: (0, 0)),
        ],
        out_shape=[
            jax.ShapeDtypeStruct((N_GRAPHS, 128), jnp.float32),
            jax.ShapeDtypeStruct((N_GRAPHS, 1), jnp.float32),
            jax.ShapeDtypeStruct((N_GRAPHS, 10), jnp.float32),
        ],
    )(*ps, *hss, dinv, W3, b3.reshape(1, 128), batch3, Wl,
      bl.reshape(1, 10))


def _final_body(sums_ref, cnt_ref, wl_ref, bl_ref, o_ref):
    pooled = sums_ref[...] / jnp.maximum(cnt_ref[...], 1.0)
    logits = (jnp.dot(pooled, wl_ref[...], preferred_element_type=jnp.float32)
              + bl_ref[...])
    m = jnp.max(logits, axis=1, keepdims=True)
    z = logits - m
    o_ref[...] = z - jnp.log(jnp.sum(jnp.exp(z), axis=1, keepdims=True))


def _tc_final(sums, cnt, Wl, bl):
    nc = Wl.shape[1]
    return pl.pallas_call(
        _final_body,
        grid=(1,),
        in_specs=[
            pl.BlockSpec((N_GRAPHS, 128), lambda i: (0, 0)),
            pl.BlockSpec((N_GRAPHS, 1), lambda i: (0, 0)),
            pl.BlockSpec((128, nc), lambda i: (0, 0)),
            pl.BlockSpec((1, nc), lambda i: (0, 0)),
        ],
        out_specs=pl.BlockSpec((N_GRAPHS, nc), lambda i: (0, 0)),
        out_shape=jax.ShapeDtypeStruct((N_GRAPHS, nc), jnp.float32),
    )(sums, cnt, Wl, bl.reshape(1, nc))


def kernel(x, edge_index, batch, W1, b1, W2, b2, W3, b3, Wl, bl):
    src = edge_index[0]
    dst = edge_index[1]
    src2 = jnp.concatenate(
        [src, jnp.zeros((E_PAD - N_EDGES,), jnp.int32)])
    dst2 = jnp.concatenate(
        [dst, jnp.full((E_PAD - N_EDGES,), N_NODES, jnp.int32)])

    ones = jnp.ones((N_PAD, 8), jnp.float32)
    (pdeg,) = _sc_prop([ones], src2, dst2, fsc=8, do_gather=False)

    xp = jnp.pad(x, ((0, N_PAD - N_NODES), (0, 3)))
    dinv, hs0 = _tc_prep(pdeg, xp)

    (p1,) = _sc_prop([hs0], src2, dst2, fsc=16)
    W1p = jnp.pad(W1, ((0, 3), (0, 0)))
    (hs1,) = _tc_layer(p1, hs0, dinv, W1p, b1, split=False)

    (p2,) = _sc_prop([hs1], src2, dst2, fsc=16)
    hs2 = _tc_layer(p2, hs1, dinv, W2, b2, split=True)

    ps = _sc_prop(list(hs2), src2, dst2, fsc=16)

    batch3 = batch.reshape(TC_GRID, 1, ROW_BLK)
    _, _, out = _tc_layer3_pool(ps, hs2, dinv, W3, b3, batch3, Wl, bl)
    return out


# speedup vs baseline: 1.1029x; 1.1029x over previous
import functools

import jax
import jax.numpy as jnp
from jax import lax
from jax.experimental import pallas as pl
from jax.experimental.pallas import tpu as pltpu
from jax.experimental.pallas import tpu_sc as plsc

N_NODES = 50000
N_EDGES = 3200000
N_GRAPHS = 512

SUB = 128
NSUB = 8
CHUNK = SUB * NSUB
NW = 32
NBUF = 4
G_ITERS_A = 120
G_ITERS_B = 76
E_PAD = 16 * (G_ITERS_A + G_ITERS_B) * CHUNK
N_PAD = 50048
ROWS_TBL = N_PAD
ROWS_PER_TILE = 3128

ROW_BLK = 2000
TC_GRID = N_NODES // ROW_BLK


def _sc_body(do_gather, nq, src1, dst1, *refs):
    hs_l = refs[:nq]
    out_l = refs[nq:2 * nq]
    idx_s, idx_d, rows, shared, sem_i, sem_g, sem_s = refs[2 * nq:]
    c = lax.axis_index("c")
    s = lax.axis_index("s")
    r0 = s * ROWS_PER_TILE
    gcur = G_ITERS_A + c * (G_ITERS_B - G_ITERS_A)
    base = (c * 16 * G_ITERS_A + s * gcur) * CHUNK

    for q in range(nq):
        hs = hs_l[q]
        pltpu.sync_copy(hs.at[pl.ds(r0, ROWS_PER_TILE)],
                        shared.at[pl.ds(r0, ROWS_PER_TILE)])
        if not do_gather:
            pltpu.sync_copy(hs.at[pl.ds(0, CHUNK)], rows.at[0])
        plsc.subcore_barrier()

        def fire_idx(g):
            b = lax.rem(g, NBUF)
            e0 = base + g * CHUNK
            pltpu.async_copy(dst1.at[pl.ds(e0, CHUNK)], idx_d.at[b],
                             sem_i.at[b])
            if do_gather:
                pltpu.async_copy(src1.at[pl.ds(e0, CHUNK)], idx_s.at[b],
                                 sem_i.at[b])

        def wait_idx(g):
            b = lax.rem(g, NBUF)
            pltpu.make_async_copy(dst1.at[pl.ds(0, CHUNK)], idx_d.at[b],
                                  sem_i.at[b]).wait()
            if do_gather:
                pltpu.make_async_copy(src1.at[pl.ds(0, CHUNK)], idx_s.at[b],
                                      sem_i.at[b]).wait()

        def fire_gather(g):
            if do_gather:
                b = lax.rem(g, NBUF)
                pltpu.async_copy(hs.at[idx_s.at[b]], rows.at[b], sem_g.at[b])

        def wait_gather(g):
            if do_gather:
                b = lax.rem(g, NBUF)
                pltpu.make_async_copy(hs.at[idx_s.at[b]], rows.at[b],
                                      sem_g.at[b]).wait()

        def fire_scatter(g):
            b = lax.rem(g, NBUF)
            rb = b if do_gather else 0
            pltpu.async_copy(rows.at[rb], shared.at[idx_d.at[b]],
                             sem_s.at[b], add=True)

        def wait_scatter(g):
            b = lax.rem(g, NBUF)
            rb = b if do_gather else 0
            pltpu.make_async_copy(rows.at[rb], shared.at[idx_d.at[b]],
                                  sem_s.at[b]).wait()

        fire_idx(0)
        fire_idx(1)

        def body(g, carry):
            wait_idx(g)
            fire_gather(g)

            @pl.when(g >= 1)
            def _():
                wait_gather(g - 1)
                fire_scatter(g - 1)

            @pl.when(g >= 2)
            def _():
                wait_scatter(g - 2)

            @pl.when(g + 2 < gcur)
            def _():
                fire_idx(g + 2)

            return carry

        lax.fori_loop(0, gcur, body, 0)
        wait_gather(gcur - 1)
        fire_scatter(gcur - 1)
        wait_scatter(gcur - 2)
        wait_scatter(gcur - 1)
        plsc.subcore_barrier()
        pltpu.sync_copy(shared.at[pl.ds(r0, ROWS_PER_TILE)],
                        out_l[q].at[c].at[pl.ds(r0, ROWS_PER_TILE)])


def _sc_prop(hs_list, src2, dst2, fsc, do_gather=True):
    nq = len(hs_list)
    mesh = plsc.VectorSubcoreMesh(core_axis_name="c", subcore_axis_name="s")
    return pl.kernel(
        functools.partial(_sc_body, do_gather, nq),
        out_type=[jax.ShapeDtypeStruct((2, ROWS_TBL, fsc), jnp.float32)] * nq,
        mesh=mesh,
        scratch_types=[
            pltpu.VMEM((NBUF, CHUNK), jnp.int32),
            pltpu.VMEM((NBUF, CHUNK), jnp.int32),
            pltpu.VMEM((NBUF, CHUNK, fsc), jnp.float32),
            pltpu.VMEM_SHARED((ROWS_TBL, fsc), jnp.float32),
            pltpu.SemaphoreType.DMA((NBUF,)),
            pltpu.SemaphoreType.DMA((NBUF,)),
            pltpu.SemaphoreType.DMA((NBUF,)),
        ],
        compiler_params=pltpu.CompilerParams(use_tc_tiling_on_sc=False),
    )(src2, dst2, *hs_list)


def _prep_body(pdeg_ref, xp_ref, dinv_ref, hs0_ref):
    deg = pdeg_ref[0, :, 0:1] + pdeg_ref[1, :, 0:1] - 1.0
    dinv = lax.rsqrt(deg)
    dinv_ref[...] = dinv
    hs0_ref[...] = xp_ref[...] * dinv


def _tc_prep(pdeg, xp):
    return pl.pallas_call(
        _prep_body,
        grid=(TC_GRID,),
        in_specs=[
            pl.BlockSpec((2, ROW_BLK, 8), lambda i: (0, i, 0)),
            pl.BlockSpec((ROW_BLK, 16), lambda i: (i, 0)),
        ],
        out_specs=[
            pl.BlockSpec((ROW_BLK, 1), lambda i: (i, 0)),
            pl.BlockSpec((ROW_BLK, 16), lambda i: (i, 0)),
        ],
        out_shape=[
            jax.ShapeDtypeStruct((N_NODES, 1), jnp.float32),
            jax.ShapeDtypeStruct((N_PAD, 16), jnp.float32),
        ],
    )(pdeg, xp)


def _layer_body(nout, p_ref, hs_ref, dinv_ref, w_ref, b_ref, *o_refs):
    dinv = dinv_ref[...]
    g = dinv * (p_ref[0] + p_ref[1] - hs_ref[...])
    h = jax.nn.relu(
        jnp.dot(g, w_ref[...], preferred_element_type=jnp.float32)
        + b_ref[...])
    hs_next = h * dinv
    if nout == 1:
        o_refs[0][...] = hs_next
    else:
        for q in range(nout):
            o_refs[q][...] = hs_next[:, 16 * q:16 * (q + 1)]


def _tc_layer(p, hs, dinv, W, b, split):
    fsc = hs.shape[1]
    fout = W.shape[1]
    if split:
        nout = fout // 16
        out_specs = [pl.BlockSpec((ROW_BLK, 16), lambda i: (i, 0))] * nout
        out_shape = [jax.ShapeDtypeStruct((N_PAD, 16), jnp.float32)] * nout
    else:
        nout = 1
        out_specs = [pl.BlockSpec((ROW_BLK, fout), lambda i: (i, 0))]
        out_shape = [jax.ShapeDtypeStruct((N_PAD, fout), jnp.float32)]
    return pl.pallas_call(
        functools.partial(_layer_body, nout),
        grid=(TC_GRID,),
        in_specs=[
            pl.BlockSpec((2, ROW_BLK, fsc), lambda i: (0, i, 0)),
            pl.BlockSpec((ROW_BLK, fsc), lambda i: (i, 0)),
            pl.BlockSpec((ROW_BLK, 1), lambda i: (i, 0)),
            pl.BlockSpec((fsc, fout), lambda i: (0, 0)),
            pl.BlockSpec((1, fout), lambda i: (0, 0)),
        ],
        out_specs=out_specs,
        out_shape=out_shape,
    )(p, hs, dinv, W, b.reshape(1, fout))


def _layer3_body(p0_ref, p1_ref, p2_ref, p3_ref, hs0_ref, hs1_ref, hs2_ref,
                 hs3_ref, dinv_ref, w_ref, b_ref,
                 batch_ref, wl_ref, bl_ref, sums_ref, cnt_ref, o_ref):
    i = pl.program_id(0)
    dinv = dinv_ref[...]
    w = w_ref[...]
    p_refs = (p0_ref, p1_ref, p2_ref, p3_ref)
    hs_refs = (hs0_ref, hs1_ref, hs2_ref, hs3_ref)
    acc = b_ref[...]
    for q in range(4):
        gq = dinv * (p_refs[q][0] + p_refs[q][1] - hs_refs[q][...])
        acc = acc + jnp.dot(gq, w[16 * q:16 * (q + 1)],
                            preferred_element_type=jnp.float32)
    h3 = jax.nn.relu(acc)
    b_ids = batch_ref[0, 0, :]
    onehot = jnp.where(
        b_ids[:, None] == lax.broadcasted_iota(jnp.int32, (ROW_BLK, N_GRAPHS), 1),
        1.0, 0.0)
    part = lax.dot_general(onehot, h3, (((0,), (0,)), ((), ())),
                           preferred_element_type=jnp.float32)
    pcnt = jnp.sum(onehot, axis=0)[:, None]

    @pl.when(i == 0)
    def _():
        sums_ref[...] = jnp.zeros_like(sums_ref)
        cnt_ref[...] = jnp.zeros_like(cnt_ref)

    sums_ref[...] += part
    cnt_ref[...] += pcnt

    @pl.when(i == TC_GRID - 1)
    def _():
        pooled = sums_ref[...] / jnp.maximum(cnt_ref[...], 1.0)
        logits = (jnp.dot(pooled, wl_ref[...],
                          preferred_element_type=jnp.float32) + bl_ref[...])
        m = jnp.max(logits, axis=1, keepdims=True)
        z = logits - m
        o_ref[...] = z - jnp.log(jnp.sum(jnp.exp(z), axis=1, keepdims=True))


def _tc_layer3_pool(ps, hss, dinv, W3, b3, batch3, Wl, bl):
    return pl.pallas_call(
        _layer3_body,
        grid=(TC_GRID,),
        in_specs=[
            pl.BlockSpec((2, ROW_BLK, 16), lambda i: (0, i, 0)),
            pl.BlockSpec((2, ROW_BLK, 16), lambda i: (0, i, 0)),
            pl.BlockSpec((2, ROW_BLK, 16), lambda i: (0, i, 0)),
            pl.BlockSpec((2, ROW_BLK, 16), lambda i: (0, i, 0)),
            pl.BlockSpec((ROW_BLK, 16), lambda i: (i, 0)),
            pl.BlockSpec((ROW_BLK, 16), lambda i: (i, 0)),
            pl.BlockSpec((ROW_BLK, 16), lambda i: (i, 0)),
            pl.BlockSpec((ROW_BLK, 16), lambda i: (i, 0)),
            pl.BlockSpec((ROW_BLK, 1), lambda i: (i, 0)),
            pl.BlockSpec((64, 128), lambda i: (0, 0)),
            pl.BlockSpec((1, 128), lambda i: (0, 0)),
            pl.BlockSpec((1, 1, ROW_BLK), lambda i: (i, 0, 0)),
            pl.BlockSpec((128, 10), lambda i: (0, 0)),
            pl.BlockSpec((1, 10), lambda i: (0, 0)),
        ],
        out_specs=[
            pl.BlockSpec((N_GRAPHS, 128), lambda i: (0, 0)),
            pl.BlockSpec((N_GRAPHS, 1), lambda i: (0, 0)),
            pl.BlockSpec((N_GRAPHS, 10), lambda i: (0, 0)),
        ],
        out_shape=[
            jax.ShapeDtypeStruct((N_GRAPHS, 128), jnp.float32),
            jax.ShapeDtypeStruct((N_GRAPHS, 1), jnp.float32),
            jax.ShapeDtypeStruct((N_GRAPHS, 10), jnp.float32),
        ],
    )(*ps, *hss, dinv, W3, b3.reshape(1, 128), batch3, Wl,
      bl.reshape(1, 10))


def _final_body(sums_ref, cnt_ref, wl_ref, bl_ref, o_ref):
    pooled = sums_ref[...] / jnp.maximum(cnt_ref[...], 1.0)
    logits = (jnp.dot(pooled, wl_ref[...], preferred_element_type=jnp.float32)
              + bl_ref[...])
    m = jnp.max(logits, axis=1, keepdims=True)
    z = logits - m
    o_ref[...] = z - jnp.log(jnp.sum(jnp.exp(z), axis=1, keepdims=True))


def _tc_final(sums, cnt, Wl, bl):
    nc = Wl.shape[1]
    return pl.pallas_call(
        _final_body,
        grid=(1,),
        in_specs=[
            pl.BlockSpec((N_GRAPHS, 128), lambda i: (0, 0)),
            pl.BlockSpec((N_GRAPHS, 1), lambda i: (0, 0)),
            pl.BlockSpec((128, nc), lambda i: (0, 0)),
            pl.BlockSpec((1, nc), lambda i: (0, 0)),
        ],
        out_specs=pl.BlockSpec((N_GRAPHS, nc), lambda i: (0, 0)),
        out_shape=jax.ShapeDtypeStruct((N_GRAPHS, nc), jnp.float32),
    )(sums, cnt, Wl, bl.reshape(1, nc))


def kernel(x, edge_index, batch, W1, b1, W2, b2, W3, b3, Wl, bl):
    src = edge_index[0]
    dst = edge_index[1]
    src2 = jnp.concatenate(
        [src, jnp.zeros((E_PAD - N_EDGES,), jnp.int32)])
    dst2 = jnp.concatenate(
        [dst, jnp.full((E_PAD - N_EDGES,), N_NODES, jnp.int32)])

    ones = jnp.ones((N_PAD, 8), jnp.float32)
    (pdeg,) = _sc_prop([ones], src2, dst2, fsc=8, do_gather=False)

    xp = jnp.pad(x, ((0, N_PAD - N_NODES), (0, 3)))
    dinv, hs0 = _tc_prep(pdeg, xp)

    (p1,) = _sc_prop([hs0], src2, dst2, fsc=16)
    W1p = jnp.pad(W1, ((0, 3), (0, 0)))
    (hs1,) = _tc_layer(p1, hs0, dinv, W1p, b1, split=False)

    (p2,) = _sc_prop([hs1], src2, dst2, fsc=16)
    hs2 = _tc_layer(p2, hs1, dinv, W2, b2, split=True)

    ps = _sc_prop(list(hs2), src2, dst2, fsc=16)

    batch3 = batch.reshape(TC_GRID, 1, ROW_BLK)
    _, _, out = _tc_layer3_pool(ps, hs2, dinv, W3, b3, batch3, Wl, bl)
    return out


# speedup vs baseline: 1.1556x; 1.0477x over previous
import functools

import jax
import jax.numpy as jnp
from jax import lax
from jax.experimental import pallas as pl
from jax.experimental.pallas import tpu as pltpu
from jax.experimental.pallas import tpu_sc as plsc

N_NODES = 50000
N_EDGES = 3200000
N_GRAPHS = 512

SUB = 128
NSUB = 8
CHUNK = SUB * NSUB
NW = 32
NBUF = 4
G_ITERS_A = 132
G_ITERS_B = 64
E_PAD = 16 * (G_ITERS_A + G_ITERS_B) * CHUNK
N_PAD = 50048
ROWS_TBL = N_PAD
ROWS_PER_TILE = 3128

ROW_BLK = 2000
TC_GRID = N_NODES // ROW_BLK


def _sc_body(do_gather, nq, src1, dst1, *refs):
    hs_l = refs[:nq]
    out_l = refs[nq:2 * nq]
    idx_s, idx_d, rows, shared, sem_i, sem_g, sem_s = refs[2 * nq:]
    c = lax.axis_index("c")
    s = lax.axis_index("s")
    r0 = s * ROWS_PER_TILE
    gcur = G_ITERS_A + c * (G_ITERS_B - G_ITERS_A)
    base = (c * 16 * G_ITERS_A + s * gcur) * CHUNK

    for q in range(nq):
        hs = hs_l[q]
        pltpu.sync_copy(hs.at[pl.ds(r0, ROWS_PER_TILE)],
                        shared.at[pl.ds(r0, ROWS_PER_TILE)])
        if not do_gather:
            pltpu.sync_copy(hs.at[pl.ds(0, CHUNK)], rows.at[0])
        plsc.subcore_barrier()

        def fire_idx(g):
            b = lax.rem(g, NBUF)
            e0 = base + g * CHUNK
            pltpu.async_copy(dst1.at[pl.ds(e0, CHUNK)], idx_d.at[b],
                             sem_i.at[b])
            if do_gather:
                pltpu.async_copy(src1.at[pl.ds(e0, CHUNK)], idx_s.at[b],
                                 sem_i.at[b])

        def wait_idx(g):
            b = lax.rem(g, NBUF)
            pltpu.make_async_copy(dst1.at[pl.ds(0, CHUNK)], idx_d.at[b],
                                  sem_i.at[b]).wait()
            if do_gather:
                pltpu.make_async_copy(src1.at[pl.ds(0, CHUNK)], idx_s.at[b],
                                      sem_i.at[b]).wait()

        def fire_gather(g):
            if do_gather:
                b = lax.rem(g, NBUF)
                pltpu.async_copy(hs.at[idx_s.at[b]], rows.at[b], sem_g.at[b])

        def wait_gather(g):
            if do_gather:
                b = lax.rem(g, NBUF)
                pltpu.make_async_copy(hs.at[idx_s.at[b]], rows.at[b],
                                      sem_g.at[b]).wait()

        def fire_scatter(g):
            b = lax.rem(g, NBUF)
            rb = b if do_gather else 0
            pltpu.async_copy(rows.at[rb], shared.at[idx_d.at[b]],
                             sem_s.at[b], add=True)

        def wait_scatter(g):
            b = lax.rem(g, NBUF)
            rb = b if do_gather else 0
            pltpu.make_async_copy(rows.at[rb], shared.at[idx_d.at[b]],
                                  sem_s.at[b]).wait()

        fire_idx(0)
        fire_idx(1)

        def body(g, carry):
            wait_idx(g)
            fire_gather(g)

            @pl.when(g >= 1)
            def _():
                wait_gather(g - 1)
                fire_scatter(g - 1)

            @pl.when(g >= 2)
            def _():
                wait_scatter(g - 2)

            @pl.when(g + 2 < gcur)
            def _():
                fire_idx(g + 2)

            return carry

        lax.fori_loop(0, gcur, body, 0)
        wait_gather(gcur - 1)
        fire_scatter(gcur - 1)
        wait_scatter(gcur - 2)
        wait_scatter(gcur - 1)
        plsc.subcore_barrier()
        pltpu.sync_copy(shared.at[pl.ds(r0, ROWS_PER_TILE)],
                        out_l[q].at[c].at[pl.ds(r0, ROWS_PER_TILE)])


def _sc_prop(hs_list, src2, dst2, fsc, do_gather=True):
    nq = len(hs_list)
    mesh = plsc.VectorSubcoreMesh(core_axis_name="c", subcore_axis_name="s")
    return pl.kernel(
        functools.partial(_sc_body, do_gather, nq),
        out_type=[jax.ShapeDtypeStruct((2, ROWS_TBL, fsc), jnp.float32)] * nq,
        mesh=mesh,
        scratch_types=[
            pltpu.VMEM((NBUF, CHUNK), jnp.int32),
            pltpu.VMEM((NBUF, CHUNK), jnp.int32),
            pltpu.VMEM((NBUF, CHUNK, fsc), jnp.float32),
            pltpu.VMEM_SHARED((ROWS_TBL, fsc), jnp.float32),
            pltpu.SemaphoreType.DMA((NBUF,)),
            pltpu.SemaphoreType.DMA((NBUF,)),
            pltpu.SemaphoreType.DMA((NBUF,)),
        ],
        compiler_params=pltpu.CompilerParams(use_tc_tiling_on_sc=False),
    )(src2, dst2, *hs_list)


def _prep_body(pdeg_ref, xp_ref, dinv_ref, hs0_ref):
    deg = pdeg_ref[0, :, 0:1] + pdeg_ref[1, :, 0:1] - 1.0
    dinv = lax.rsqrt(deg)
    dinv_ref[...] = dinv
    hs0_ref[...] = xp_ref[...] * dinv


def _tc_prep(pdeg, xp):
    return pl.pallas_call(
        _prep_body,
        grid=(TC_GRID,),
        in_specs=[
            pl.BlockSpec((2, ROW_BLK, 8), lambda i: (0, i, 0)),
            pl.BlockSpec((ROW_BLK, 16), lambda i: (i, 0)),
        ],
        out_specs=[
            pl.BlockSpec((ROW_BLK, 1), lambda i: (i, 0)),
            pl.BlockSpec((ROW_BLK, 16), lambda i: (i, 0)),
        ],
        out_shape=[
            jax.ShapeDtypeStruct((N_NODES, 1), jnp.float32),
            jax.ShapeDtypeStruct((N_PAD, 16), jnp.float32),
        ],
    )(pdeg, xp)


def _layer_body(nout, p_ref, hs_ref, dinv_ref, w_ref, b_ref, *o_refs):
    dinv = dinv_ref[...]
    g = dinv * (p_ref[0] + p_ref[1] - hs_ref[...])
    h = jax.nn.relu(
        jnp.dot(g, w_ref[...], preferred_element_type=jnp.float32)
        + b_ref[...])
    hs_next = h * dinv
    if nout == 1:
        o_refs[0][...] = hs_next
    else:
        for q in range(nout):
            o_refs[q][...] = hs_next[:, 16 * q:16 * (q + 1)]


def _tc_layer(p, hs, dinv, W, b, split):
    fsc = hs.shape[1]
    fout = W.shape[1]
    if split:
        nout = fout // 16
        out_specs = [pl.BlockSpec((ROW_BLK, 16), lambda i: (i, 0))] * nout
        out_shape = [jax.ShapeDtypeStruct((N_PAD, 16), jnp.float32)] * nout
    else:
        nout = 1
        out_specs = [pl.BlockSpec((ROW_BLK, fout), lambda i: (i, 0))]
        out_shape = [jax.ShapeDtypeStruct((N_PAD, fout), jnp.float32)]
    return pl.pallas_call(
        functools.partial(_layer_body, nout),
        grid=(TC_GRID,),
        in_specs=[
            pl.BlockSpec((2, ROW_BLK, fsc), lambda i: (0, i, 0)),
            pl.BlockSpec((ROW_BLK, fsc), lambda i: (i, 0)),
            pl.BlockSpec((ROW_BLK, 1), lambda i: (i, 0)),
            pl.BlockSpec((fsc, fout), lambda i: (0, 0)),
            pl.BlockSpec((1, fout), lambda i: (0, 0)),
        ],
        out_specs=out_specs,
        out_shape=out_shape,
    )(p, hs, dinv, W, b.reshape(1, fout))


def _layer3_body(p0_ref, p1_ref, p2_ref, p3_ref, hs0_ref, hs1_ref, hs2_ref,
                 hs3_ref, dinv_ref, w_ref, b_ref,
                 batch_ref, wl_ref, bl_ref, sums_ref, cnt_ref, o_ref):
    i = pl.program_id(0)
    dinv = dinv_ref[...]
    w = w_ref[...]
    p_refs = (p0_ref, p1_ref, p2_ref, p3_ref)
    hs_refs = (hs0_ref, hs1_ref, hs2_ref, hs3_ref)
    acc = b_ref[...]
    for q in range(4):
        gq = dinv * (p_refs[q][0] + p_refs[q][1] - hs_refs[q][...])
        acc = acc + jnp.dot(gq, w[16 * q:16 * (q + 1)],
                            preferred_element_type=jnp.float32)
    h3 = jax.nn.relu(acc)
    b_ids = batch_ref[0, 0, :]
    onehot = jnp.where(
        b_ids[:, None] == lax.broadcasted_iota(jnp.int32, (ROW_BLK, N_GRAPHS), 1),
        1.0, 0.0)
    part = lax.dot_general(onehot, h3, (((0,), (0,)), ((), ())),
                           preferred_element_type=jnp.float32)
    pcnt = jnp.sum(onehot, axis=0)[:, None]

    @pl.when(i == 0)
    def _():
        sums_ref[...] = jnp.zeros_like(sums_ref)
        cnt_ref[...] = jnp.zeros_like(cnt_ref)

    sums_ref[...] += part
    cnt_ref[...] += pcnt

    @pl.when(i == TC_GRID - 1)
    def _():
        pooled = sums_ref[...] / jnp.maximum(cnt_ref[...], 1.0)
        logits = (jnp.dot(pooled, wl_ref[...],
                          preferred_element_type=jnp.float32) + bl_ref[...])
        m = jnp.max(logits, axis=1, keepdims=True)
        z = logits - m
        o_ref[...] = z - jnp.log(jnp.sum(jnp.exp(z), axis=1, keepdims=True))


def _tc_layer3_pool(ps, hss, dinv, W3, b3, batch3, Wl, bl):
    return pl.pallas_call(
        _layer3_body,
        grid=(TC_GRID,),
        in_specs=[
            pl.BlockSpec((2, ROW_BLK, 16), lambda i: (0, i, 0)),
            pl.BlockSpec((2, ROW_BLK, 16), lambda i: (0, i, 0)),
            pl.BlockSpec((2, ROW_BLK, 16), lambda i: (0, i, 0)),
            pl.BlockSpec((2, ROW_BLK, 16), lambda i: (0, i, 0)),
            pl.BlockSpec((ROW_BLK, 16), lambda i: (i, 0)),
            pl.BlockSpec((ROW_BLK, 16), lambda i: (i, 0)),
            pl.BlockSpec((ROW_BLK, 16), lambda i: (i, 0)),
            pl.BlockSpec((ROW_BLK, 16), lambda i: (i, 0)),
            pl.BlockSpec((ROW_BLK, 1), lambda i: (i, 0)),
            pl.BlockSpec((64, 128), lambda i: (0, 0)),
            pl.BlockSpec((1, 128), lambda i: (0, 0)),
            pl.BlockSpec((1, 1, ROW_BLK), lambda i: (i, 0, 0)),
            pl.BlockSpec((128, 10), lambda i: (0, 0)),
            pl.BlockSpec((1, 10), lambda i: (0, 0)),
        ],
        out_specs=[
            pl.BlockSpec((N_GRAPHS, 128), lambda i: (0, 0)),
            pl.BlockSpec((N_GRAPHS, 1), lambda i: (0, 0)),
            pl.BlockSpec((N_GRAPHS, 10), lambda i: (0, 0)),
        ],
        out_shape=[
            jax.ShapeDtypeStruct((N_GRAPHS, 128), jnp.float32),
            jax.ShapeDtypeStruct((N_GRAPHS, 1), jnp.float32),
            jax.ShapeDtypeStruct((N_GRAPHS, 10), jnp.float32),
        ],
    )(*ps, *hss, dinv, W3, b3.reshape(1, 128), batch3, Wl,
      bl.reshape(1, 10))


def _final_body(sums_ref, cnt_ref, wl_ref, bl_ref, o_ref):
    pooled = sums_ref[...] / jnp.maximum(cnt_ref[...], 1.0)
    logits = (jnp.dot(pooled, wl_ref[...], preferred_element_type=jnp.float32)
              + bl_ref[...])
    m = jnp.max(logits, axis=1, keepdims=True)
    z = logits - m
    o_ref[...] = z - jnp.log(jnp.sum(jnp.exp(z), axis=1, keepdims=True))


def _tc_final(sums, cnt, Wl, bl):
    nc = Wl.shape[1]
    return pl.pallas_call(
        _final_body,
        grid=(1,),
        in_specs=[
            pl.BlockSpec((N_GRAPHS, 128), lambda i: (0, 0)),
            pl.BlockSpec((N_GRAPHS, 1), lambda i: (0, 0)),
            pl.BlockSpec((128, nc), lambda i: (0, 0)),
            pl.BlockSpec((1, nc), lambda i: (0, 0)),
        ],
        out_specs=pl.BlockSpec((N_GRAPHS, nc), lambda i: (0, 0)),
        out_shape=jax.ShapeDtypeStruct((N_GRAPHS, nc), jnp.float32),
    )(sums, cnt, Wl, bl.reshape(1, nc))


def kernel(x, edge_index, batch, W1, b1, W2, b2, W3, b3, Wl, bl):
    src = edge_index[0]
    dst = edge_index[1]
    src2 = jnp.concatenate(
        [src, jnp.zeros((E_PAD - N_EDGES,), jnp.int32)])
    dst2 = jnp.concatenate(
        [dst, jnp.full((E_PAD - N_EDGES,), N_NODES, jnp.int32)])

    ones = jnp.ones((N_PAD, 8), jnp.float32)
    (pdeg,) = _sc_prop([ones], src2, dst2, fsc=8, do_gather=False)

    xp = jnp.pad(x, ((0, N_PAD - N_NODES), (0, 3)))
    dinv, hs0 = _tc_prep(pdeg, xp)

    (p1,) = _sc_prop([hs0], src2, dst2, fsc=16)
    W1p = jnp.pad(W1, ((0, 3), (0, 0)))
    (hs1,) = _tc_layer(p1, hs0, dinv, W1p, b1, split=False)

    (p2,) = _sc_prop([hs1], src2, dst2, fsc=16)
    hs2 = _tc_layer(p2, hs1, dinv, W2, b2, split=True)

    ps = _sc_prop(list(hs2), src2, dst2, fsc=16)

    batch3 = batch.reshape(TC_GRID, 1, ROW_BLK)
    _, _, out = _tc_layer3_pool(ps, hs2, dinv, W3, b3, batch3, Wl, bl)
    return out
